# baseline (speedup 1.0000x reference)
import functools

import jax
import jax.numpy as jnp
from jax import lax
from jax.experimental import pallas as pl
from jax.experimental.pallas import tpu as pltpu
from jax.experimental.pallas import tpu_sc as plsc

B = 16384
D = 128
NC = 2
NS = 16
NW = NC * NS
BPW = B // NW
CHUNK = 128
NCHUNK = BPW // CHUNK

def _sc_gather(emb, ctx, src_idx, dst_idx):
    mesh = plsc.VectorSubcoreMesh(core_axis_name="c", subcore_axis_name="s")
    out_t = jax.ShapeDtypeStruct((B, D), jnp.float32)

    @functools.partial(
        pl.kernel,
        out_type=(out_t, out_t),
        mesh=mesh,
        scratch_types=[
            pltpu.VMEM((BPW,), jnp.int32),
            pltpu.VMEM((BPW,), jnp.int32),
            pltpu.VMEM((CHUNK, D), jnp.float32),
            pltpu.VMEM((CHUNK, D), jnp.float32),
            pltpu.SemaphoreType.DMA,
            pltpu.SemaphoreType.DMA,
        ],
    )
    def k(emb_hbm, ctx_hbm, src_hbm, dst_hbm, osrc_hbm, odst_hbm,
          sidx_v, didx_v, srows_v, drows_v, sem_s, sem_d):
        wid = lax.axis_index("s") * NC + lax.axis_index("c")
        base = wid * BPW
        pltpu.sync_copy(src_hbm.at[pl.ds(base, BPW)], sidx_v)
        pltpu.sync_copy(dst_hbm.at[pl.ds(base, BPW)], didx_v)
        for c in range(NCHUNK):
            off = c * CHUNK
            pltpu.async_copy(
                emb_hbm.at[sidx_v.at[pl.ds(off, CHUNK)]], srows_v, sem_s
            ).wait()
            pltpu.sync_copy(srows_v, osrc_hbm.at[pl.ds(base + off, CHUNK)])
            pltpu.async_copy(
                ctx_hbm.at[didx_v.at[pl.ds(off, CHUNK)]], drows_v, sem_d
            ).wait()
            pltpu.sync_copy(drows_v, odst_hbm.at[pl.ds(base + off, CHUNK)])

    return k(emb, ctx, src_idx, dst_idx)


_TC_BLK = 2048


def _tc_loss_body(a_ref, b_ref, o_ref):
    i = pl.program_id(0)

    @pl.when(i == 0)
    def _():
        o_ref[...] = jnp.zeros_like(o_ref)

    dot = jnp.sum(a_ref[...] * b_ref[...], axis=1)
    ls = -jax.nn.softplus(-dot)
    o_ref[...] += jnp.sum(ls).reshape(1, 1)

    @pl.when(i == (B // _TC_BLK) - 1)
    def _():
        o_ref[...] = -o_ref[...] * (1.0 / B)


def _tc_loss(a, b):
    out = pl.pallas_call(
        _tc_loss_body,
        grid=(B // _TC_BLK,),
        in_specs=[
            pl.BlockSpec((_TC_BLK, D), lambda i: (i, 0)),
            pl.BlockSpec((_TC_BLK, D), lambda i: (i, 0)),
        ],
        out_specs=pl.BlockSpec((1, 1), lambda i: (0, 0)),
        out_shape=jax.ShapeDtypeStruct((1, 1), jnp.float32),
    )(a, b)
    return out[0, 0]


def kernel(src_nodes, dst_nodes, embedding, context_embedding):
    src_e, dst_e = _sc_gather(
        embedding,
        context_embedding,
        src_nodes.astype(jnp.int32),
        dst_nodes.astype(jnp.int32),
    )
    return _tc_loss(src_e, dst_e)


# speedup vs baseline: 1.0768x; 1.0768x over previous
import functools

import jax
import jax.numpy as jnp
from jax import lax
from jax.experimental import pallas as pl
from jax.experimental.pallas import tpu as pltpu
from jax.experimental.pallas import tpu_sc as plsc

B = 16384
D = 128
L = 16
NC = 2
NS = 16
NW = NC * NS
BPW = B // NW
CHUNK = 64
NCHUNK = BPW // CHUNK
ROW_UNROLL = 4


def _sc_gather_dot(emb, ctx, src_idx, dst_idx):
    mesh = plsc.VectorSubcoreMesh(core_axis_name="c", subcore_axis_name="s")
    out_t = jax.ShapeDtypeStruct((B, L), jnp.float32)

    @functools.partial(
        pl.kernel,
        out_type=out_t,
        mesh=mesh,
        scratch_types=[
            pltpu.VMEM((BPW,), jnp.int32),
            pltpu.VMEM((BPW,), jnp.int32),
            pltpu.VMEM((CHUNK, D), jnp.float32),
            pltpu.VMEM((CHUNK, D), jnp.float32),
            pltpu.VMEM((CHUNK, D), jnp.float32),
            pltpu.VMEM((CHUNK, D), jnp.float32),
            pltpu.VMEM((BPW, L), jnp.float32),
            pltpu.SemaphoreType.DMA,
            pltpu.SemaphoreType.DMA,
            pltpu.SemaphoreType.DMA,
            pltpu.SemaphoreType.DMA,
        ],
    )
    def k(emb_hbm, ctx_hbm, src_hbm, dst_hbm, out_hbm,
          sidx_v, didx_v, sbuf0, sbuf1, dbuf0, dbuf1, part_v,
          sem_s0, sem_s1, sem_d0, sem_d1):
        wid = lax.axis_index("s") * NC + lax.axis_index("c")
        base = wid * BPW
        pltpu.sync_copy(src_hbm.at[pl.ds(base, BPW)], sidx_v)
        pltpu.sync_copy(dst_hbm.at[pl.ds(base, BPW)], didx_v)

        sbufs = (sbuf0, sbuf1)
        dbufs = (dbuf0, dbuf1)
        ssems = (sem_s0, sem_s1)
        dsems = (sem_d0, sem_d1)

        def start(c):
            off = c * CHUNK
            cs = pltpu.async_copy(
                emb_hbm.at[sidx_v.at[pl.ds(off, CHUNK)]], sbufs[c % 2],
                ssems[c % 2])
            cd = pltpu.async_copy(
                ctx_hbm.at[didx_v.at[pl.ds(off, CHUNK)]], dbufs[c % 2],
                dsems[c % 2])
            return cs, cd

        pend = start(0)
        for c in range(NCHUNK):
            nxt = start(c + 1) if c + 1 < NCHUNK else None
            cs, cd = pend
            cs.wait()
            cd.wait()
            sb = sbufs[c % 2]
            db = dbufs[c % 2]
            pbase = c * CHUNK

            @pl.loop(0, CHUNK, step=ROW_UNROLL)
            def _(r0):
                for u in range(ROW_UNROLL):
                    r = r0 + u
                    acc = sb[r, pl.ds(0, L)] * db[r, pl.ds(0, L)]
                    for g in range(1, D // L):
                        acc += sb[r, pl.ds(g * L, L)] * db[r, pl.ds(g * L, L)]
                    part_v[pbase + r, :] = acc

            pend = nxt

        pltpu.sync_copy(part_v, out_hbm.at[pl.ds(base, BPW)])

    return k(emb, ctx, src_idx, dst_idx)


def _tc_loss_body(p_ref, o_ref):
    dot = jnp.sum(p_ref[...], axis=1)
    ls = -jax.nn.softplus(-dot)
    o_ref[...] = (-jnp.sum(ls) * (1.0 / B)).reshape(1, 1)


def _tc_loss(p):
    out = pl.pallas_call(
        _tc_loss_body,
        out_shape=jax.ShapeDtypeStruct((1, 1), jnp.float32),
    )(p)
    return out[0, 0]


def kernel(src_nodes, dst_nodes, embedding, context_embedding):
    part = _sc_gather_dot(
        embedding,
        context_embedding,
        src_nodes.astype(jnp.int32),
        dst_nodes.astype(jnp.int32),
    )
    return _tc_loss(part)


# speedup vs baseline: 1.2275x; 1.1399x over previous
import functools

import jax
import jax.numpy as jnp
from jax import lax
from jax.experimental import pallas as pl
from jax.experimental.pallas import tpu as pltpu
from jax.experimental.pallas import tpu_sc as plsc

B = 16384
D = 128
L = 16
NC = 2
NS = 16
NW = NC * NS
BPW = B // NW
CHUNK = 64
NCHUNK = BPW // CHUNK
ROW_UNROLL = 8
PCOLS = 128
PROWS = B * L // PCOLS
PROWS_PW = BPW * L // PCOLS


def _sc_gather_dot(emb, ctx, src_idx, dst_idx):
    mesh = plsc.VectorSubcoreMesh(core_axis_name="c", subcore_axis_name="s")
    out_t = jax.ShapeDtypeStruct((PROWS, PCOLS), jnp.float32)

    @functools.partial(
        pl.kernel,
        out_type=out_t,
        mesh=mesh,
        scratch_types=[
            pltpu.VMEM((BPW,), jnp.int32),
            pltpu.VMEM((BPW,), jnp.int32),
            pltpu.VMEM((CHUNK, D), jnp.float32),
            pltpu.VMEM((CHUNK, D), jnp.float32),
            pltpu.VMEM((CHUNK, D), jnp.float32),
            pltpu.VMEM((CHUNK, D), jnp.float32),
            pltpu.VMEM((PROWS_PW, PCOLS), jnp.float32),
            pltpu.SemaphoreType.DMA,
            pltpu.SemaphoreType.DMA,
            pltpu.SemaphoreType.DMA,
            pltpu.SemaphoreType.DMA,
        ],
    )
    def k(emb_hbm, ctx_hbm, src_hbm, dst_hbm, out_hbm,
          sidx_v, didx_v, sbuf0, sbuf1, dbuf0, dbuf1, part_v,
          sem_s0, sem_s1, sem_d0, sem_d1):
        wid = lax.axis_index("s") * NC + lax.axis_index("c")
        base = wid * BPW
        pltpu.sync_copy(src_hbm.at[pl.ds(base, BPW)], sidx_v)
        pltpu.sync_copy(dst_hbm.at[pl.ds(base, BPW)], didx_v)

        sbufs = (sbuf0, sbuf1)
        dbufs = (dbuf0, dbuf1)
        ssems = (sem_s0, sem_s1)
        dsems = (sem_d0, sem_d1)

        def start(c):
            off = c * CHUNK
            cs = pltpu.async_copy(
                emb_hbm.at[sidx_v.at[pl.ds(off, CHUNK)]], sbufs[c % 2],
                ssems[c % 2])
            cd = pltpu.async_copy(
                ctx_hbm.at[didx_v.at[pl.ds(off, CHUNK)]], dbufs[c % 2],
                dsems[c % 2])
            return cs, cd

        pend = start(0)
        for c in range(NCHUNK):
            nxt = start(c + 1) if c + 1 < NCHUNK else None
            cs, cd = pend
            cs.wait()
            cd.wait()
            sb = sbufs[c % 2]
            db = dbufs[c % 2]
            pbase = c * CHUNK

            @pl.loop(0, CHUNK, step=ROW_UNROLL)
            def _(r0):
                pline = (pbase + r0) // ROW_UNROLL
                for u in range(ROW_UNROLL):
                    r = r0 + u
                    acc = sb[r, pl.ds(0, L)] * db[r, pl.ds(0, L)]
                    for g in range(1, D // L):
                        acc += sb[r, pl.ds(g * L, L)] * db[r, pl.ds(g * L, L)]
                    part_v[pline, pl.ds(u * L, L)] = acc

            pend = nxt

        pltpu.sync_copy(part_v, out_hbm.at[pl.ds(wid * PROWS_PW, PROWS_PW)])

    return k(emb, ctx, src_idx, dst_idx)


def _tc_loss_body(p_ref, o_ref):
    y = p_ref[...]
    r_grp = jax.lax.broadcasted_iota(jnp.int32, (PCOLS, PCOLS), 0) // L
    c_grp = jax.lax.broadcasted_iota(jnp.int32, (PCOLS, PCOLS), 1) // L
    g = (r_grp == c_grp).astype(jnp.float32)
    dot = jax.lax.dot_general(y, g, (((1,), (0,)), ((), ())),
                              preferred_element_type=jnp.float32)
    sp = jax.nn.softplus(-dot)
    o_ref[...] = (jnp.sum(sp) * (1.0 / (B * L))).reshape(1, 1)


def _tc_loss(p):
    out = pl.pallas_call(
        _tc_loss_body,
        out_shape=jax.ShapeDtypeStruct((1, 1), jnp.float32),
    )(p)
    return out[0, 0]


def kernel(src_nodes, dst_nodes, embedding, context_embedding):
    part = _sc_gather_dot(
        embedding,
        context_embedding,
        src_nodes.astype(jnp.int32),
        dst_nodes.astype(jnp.int32),
    )
    return _tc_loss(part)


# speedup vs baseline: 1.3028x; 1.0614x over previous
import functools

import jax
import jax.numpy as jnp
from jax import lax
from jax.experimental import pallas as pl
from jax.experimental.pallas import tpu as pltpu
from jax.experimental.pallas import tpu_sc as plsc

B = 16384
D = 128
L = 16
NC = 2
NS = 16
NW = NC * NS
BPW = B // NW
CHUNK = 64
NCHUNK = BPW // CHUNK
ROW_UNROLL = 8
PCOLS = 128
PROWS = B * L // PCOLS
PROWS_PW = BPW * L // PCOLS


def _sc_gather_dot(emb, ctx, src_idx, dst_idx):
    mesh = plsc.VectorSubcoreMesh(core_axis_name="c", subcore_axis_name="s")
    out_t = jax.ShapeDtypeStruct((PROWS, PCOLS), jnp.float32)

    @functools.partial(
        pl.kernel,
        out_type=out_t,
        mesh=mesh,
        scratch_types=[
            pltpu.VMEM((BPW,), jnp.int32),
            pltpu.VMEM((BPW,), jnp.int32),
            pltpu.VMEM((CHUNK, D), jnp.float32),
            pltpu.VMEM((CHUNK, D), jnp.float32),
            pltpu.VMEM((CHUNK, D), jnp.float32),
            pltpu.VMEM((CHUNK, D), jnp.float32),
            pltpu.VMEM((PROWS_PW, PCOLS), jnp.float32),
            pltpu.SemaphoreType.DMA,
            pltpu.SemaphoreType.DMA,
            pltpu.SemaphoreType.DMA,
            pltpu.SemaphoreType.DMA,
        ],
    )
    def k(emb_hbm, ctx_hbm, src_hbm, dst_hbm, out_hbm,
          sidx_v, didx_v, sbuf0, sbuf1, dbuf0, dbuf1, part_v,
          sem_s0, sem_s1, sem_d0, sem_d1):
        wid = lax.axis_index("s") * NC + lax.axis_index("c")
        base = wid * BPW
        pltpu.sync_copy(src_hbm.at[pl.ds(base, BPW)], sidx_v)
        pltpu.sync_copy(dst_hbm.at[pl.ds(base, BPW)], didx_v)

        sbufs = (sbuf0, sbuf1)
        dbufs = (dbuf0, dbuf1)
        ssems = (sem_s0, sem_s1)
        dsems = (sem_d0, sem_d1)

        def start(c):
            off = c * CHUNK
            cs = pltpu.async_copy(
                emb_hbm.at[sidx_v.at[pl.ds(off, CHUNK)]], sbufs[c % 2],
                ssems[c % 2])
            cd = pltpu.async_copy(
                ctx_hbm.at[didx_v.at[pl.ds(off, CHUNK)]], dbufs[c % 2],
                dsems[c % 2])
            return cs, cd

        pend = start(0)
        for c in range(NCHUNK):
            nxt = start(c + 1) if c + 1 < NCHUNK else None
            cs, cd = pend
            cs.wait()
            cd.wait()
            sb = sbufs[c % 2]
            db = dbufs[c % 2]
            pbase = c * CHUNK

            @plsc.parallel_loop(0, CHUNK, step=ROW_UNROLL, unroll=2)
            def _(r0):
                pline = (pbase + r0) // ROW_UNROLL
                for u in range(ROW_UNROLL):
                    r = r0 + u
                    acc = sb[r, pl.ds(0, L)] * db[r, pl.ds(0, L)]
                    for g in range(1, D // L):
                        acc += sb[r, pl.ds(g * L, L)] * db[r, pl.ds(g * L, L)]
                    part_v[pline, pl.ds(u * L, L)] = acc

            pend = nxt

        pltpu.sync_copy(part_v, out_hbm.at[pl.ds(wid * PROWS_PW, PROWS_PW)])

    return k(emb, ctx, src_idx, dst_idx)


def _tc_loss_body(p_ref, o_ref):
    y = p_ref[...]
    r_grp = jax.lax.broadcasted_iota(jnp.int32, (PCOLS, PCOLS), 0) // L
    c_grp = jax.lax.broadcasted_iota(jnp.int32, (PCOLS, PCOLS), 1) // L
    g = (r_grp == c_grp).astype(jnp.float32)
    dot = jax.lax.dot_general(y, g, (((1,), (0,)), ((), ())),
                              preferred_element_type=jnp.float32)
    sp = jax.nn.softplus(-dot)
    o_ref[...] = (jnp.sum(sp) * (1.0 / (B * L))).reshape(1, 1)


def _tc_loss(p):
    out = pl.pallas_call(
        _tc_loss_body,
        out_shape=jax.ShapeDtypeStruct((1, 1), jnp.float32),
    )(p)
    return out[0, 0]


def kernel(src_nodes, dst_nodes, embedding, context_embedding):
    part = _sc_gather_dot(
        embedding,
        context_embedding,
        src_nodes.astype(jnp.int32),
        dst_nodes.astype(jnp.int32),
    )
    return _tc_loss(part)


# speedup vs baseline: 1.3596x; 1.0436x over previous
import functools

import jax
import jax.numpy as jnp
from jax import lax
from jax.experimental import pallas as pl
from jax.experimental.pallas import tpu as pltpu
from jax.experimental.pallas import tpu_sc as plsc

B = 16384
D = 128
L = 16
NC = 2
NS = 16
NW = NC * NS
BPW = B // NW
CHUNK = 64
NCHUNK = BPW // CHUNK
ROW_UNROLL = 8
PCOLS = 128
PROWS = B * L // PCOLS
PROWS_PW = BPW * L // PCOLS


def _sc_gather_dot(emb, ctx, src_idx, dst_idx):
    mesh = plsc.VectorSubcoreMesh(core_axis_name="c", subcore_axis_name="s")
    out_t = jax.ShapeDtypeStruct((PROWS, PCOLS), jnp.float32)

    @functools.partial(
        pl.kernel,
        out_type=out_t,
        mesh=mesh,
        scratch_types=[
            pltpu.VMEM((BPW,), jnp.int32),
            pltpu.VMEM((BPW,), jnp.int32),
            pltpu.VMEM((CHUNK, D), jnp.float32),
            pltpu.VMEM((CHUNK, D), jnp.float32),
            pltpu.VMEM((CHUNK, D), jnp.float32),
            pltpu.VMEM((CHUNK, D), jnp.float32),
            pltpu.VMEM((CHUNK, D), jnp.float32),
            pltpu.VMEM((CHUNK, D), jnp.float32),
            pltpu.VMEM((PROWS_PW, PCOLS), jnp.float32),
            pltpu.SemaphoreType.DMA,
            pltpu.SemaphoreType.DMA,
            pltpu.SemaphoreType.DMA,
            pltpu.SemaphoreType.DMA,
            pltpu.SemaphoreType.DMA,
            pltpu.SemaphoreType.DMA,
        ],
    )
    def k(emb_hbm, ctx_hbm, src_hbm, dst_hbm, out_hbm,
          sidx_v, didx_v, sbuf0, sbuf1, sbuf2, dbuf0, dbuf1, dbuf2, part_v,
          sem_s0, sem_s1, sem_s2, sem_d0, sem_d1, sem_d2):
        wid = lax.axis_index("s") * NC + lax.axis_index("c")
        base = wid * BPW
        pltpu.sync_copy(src_hbm.at[pl.ds(base, BPW)], sidx_v)
        pltpu.sync_copy(dst_hbm.at[pl.ds(base, BPW)], didx_v)

        sbufs = (sbuf0, sbuf1, sbuf2)
        dbufs = (dbuf0, dbuf1, dbuf2)
        ssems = (sem_s0, sem_s1, sem_s2)
        dsems = (sem_d0, sem_d1, sem_d2)
        DEPTH = 3

        def start(c):
            off = c * CHUNK
            cs = pltpu.async_copy(
                emb_hbm.at[sidx_v.at[pl.ds(off, CHUNK)]], sbufs[c % DEPTH],
                ssems[c % DEPTH])
            cd = pltpu.async_copy(
                ctx_hbm.at[didx_v.at[pl.ds(off, CHUNK)]], dbufs[c % DEPTH],
                dsems[c % DEPTH])
            return cs, cd

        pend = [start(c) for c in range(DEPTH)]
        for c in range(NCHUNK):
            cs, cd = pend[c % DEPTH]
            cs.wait()
            cd.wait()
            sb = sbufs[c % DEPTH]
            db = dbufs[c % DEPTH]
            pbase = c * CHUNK

            @plsc.parallel_loop(0, CHUNK, step=ROW_UNROLL, unroll=2)
            def _(r0):
                pline = (pbase + r0) // ROW_UNROLL
                for u in range(ROW_UNROLL):
                    r = r0 + u
                    acc = sb[r, pl.ds(0, L)] * db[r, pl.ds(0, L)]
                    for g in range(1, D // L):
                        acc += sb[r, pl.ds(g * L, L)] * db[r, pl.ds(g * L, L)]
                    part_v[pline, pl.ds(u * L, L)] = acc

            if c + DEPTH < NCHUNK:
                pend[c % DEPTH] = start(c + DEPTH)

        pltpu.sync_copy(part_v, out_hbm.at[pl.ds(wid * PROWS_PW, PROWS_PW)])

    return k(emb, ctx, src_idx, dst_idx)


def _tc_loss_body(p_ref, o_ref):
    y = p_ref[...]
    r_grp = jax.lax.broadcasted_iota(jnp.int32, (PCOLS, PCOLS), 0) // L
    c_grp = jax.lax.broadcasted_iota(jnp.int32, (PCOLS, PCOLS), 1) // L
    g = (r_grp == c_grp).astype(jnp.float32)
    dot = jax.lax.dot_general(y, g, (((1,), (0,)), ((), ())),
                              preferred_element_type=jnp.float32)
    sp = jax.nn.softplus(-dot)
    o_ref[...] = (jnp.sum(sp) * (1.0 / (B * L))).reshape(1, 1)


def _tc_loss(p):
    out = pl.pallas_call(
        _tc_loss_body,
        out_shape=jax.ShapeDtypeStruct((1, 1), jnp.float32),
    )(p)
    return out[0, 0]


def kernel(src_nodes, dst_nodes, embedding, context_embedding):
    part = _sc_gather_dot(
        embedding,
        context_embedding,
        src_nodes.astype(jnp.int32),
        dst_nodes.astype(jnp.int32),
    )
    return _tc_loss(part)
